# Initial kernel scaffold; baseline (speedup 1.0000x reference)
import jax
import jax.numpy as jnp
from jax.experimental import pallas as pl


def kernel(x, edge_index, edge_vals, W):
    raise NotImplementedError("write your pallas kernel here")


# speedup vs baseline: 5.7989x; 5.7989x over previous
import functools

import jax
import jax.numpy as jnp
from jax import lax
from jax.experimental import pallas as pl
from jax.experimental.pallas import tpu as pltpu
from jax.experimental.pallas import tpu_sc as plsc

_N = 10000
_E = 640000
_D = 128
_HOPS = 3
_NC = 2
_NS = 16
_NW = _NC * _NS
_C = 128
_TOTAL_E = _HOPS * _E
_CPW = -(-_TOTAL_E // (_NW * _C))
_EPAD = _NW * _CPW * _C
_RPT = _N // _NS
_MB = 2000


def _matmul_body(x_ref, w_ref, out_ref):
    out_ref[0] = jnp.dot(x_ref[...], w_ref[0],
                         preferred_element_type=jnp.float32)


def _feats(x, W):
    return pl.pallas_call(
        _matmul_body,
        grid=(_HOPS, _N // _MB),
        in_specs=[
            pl.BlockSpec((_MB, _D), lambda h, i: (i, 0)),
            pl.BlockSpec((1, _D, _D), lambda h, i: (h, 0, 0)),
        ],
        out_specs=pl.BlockSpec((1, _MB, _D), lambda h, i: (h, i, 0)),
        out_shape=jax.ShapeDtypeStruct((_HOPS, _N, _D), jnp.float32),
    )(x, W)


def _combine_body(p_ref, o_ref):
    o_ref[...] = jnp.maximum(p_ref[0] + p_ref[1], 0.0)


def _combine(p):
    return pl.pallas_call(
        _combine_body,
        grid=(_N // _MB,),
        in_specs=[pl.BlockSpec((_NC, _MB, _D), lambda i: (0, i, 0))],
        out_specs=pl.BlockSpec((_MB, _D), lambda i: (i, 0)),
        out_shape=jax.ShapeDtypeStruct((_N, _D), jnp.float32),
    )(p)


@functools.partial(
    pl.kernel,
    out_type=jax.ShapeDtypeStruct((_NC, _N, _D), jnp.float32),
    mesh=plsc.VectorSubcoreMesh(core_axis_name="c", subcore_axis_name="s"),
    scratch_types=[
        pltpu.VMEM((_C,), jnp.int32),
        pltpu.VMEM((_C,), jnp.int32),
        pltpu.VMEM((_C,), jnp.float32),
        pltpu.VMEM((_C, _D), jnp.float32),
        pltpu.VMEM_SHARED((_N, _D), jnp.float32),
        pltpu.SemaphoreType.DMA,
    ],
)
def _propagate(feat_hbm, col_hbm, row_hbm, val_hbm, out_hbm,
               col_v, row_v, val_v, rows_v, accum, sem):
    c = lax.axis_index("c")
    s = lax.axis_index("s")
    w = s * _NC + c

    def _zrow(e, carry):
        for k in range(_D // 16):
            rows_v[e, pl.ds(k * 16, 16)] = jnp.zeros((16,), jnp.float32)
        return carry

    lax.fori_loop(0, _C, _zrow, 0)

    nspan = _N // _C
    tail = _N - nspan * _C
    for j in range((nspan + _NS - 1) // _NS):
        idx = s + _NS * j

        @pl.when(idx < nspan)
        def _zero_span():
            off = pl.multiple_of(idx * _C, 8)
            pltpu.sync_copy(rows_v, accum.at[pl.ds(off, _C)])

    @pl.when(s == _NS - 1)
    def _zero_tail():
        pltpu.sync_copy(rows_v.at[pl.ds(0, tail)],
                        accum.at[pl.ds(nspan * _C, tail)])

    plsc.subcore_barrier()

    def _chunk(i, carry):
        g = w * _CPW + i
        pltpu.sync_copy(col_hbm.at[g], col_v)
        pltpu.sync_copy(row_hbm.at[g], row_v)
        pltpu.sync_copy(val_hbm.at[g], val_v)
        pltpu.async_copy(feat_hbm.at[col_v], rows_v, sem).wait()

        def _scale(g, inner):
            v16 = val_v[pl.ds(g * 16, 16)]
            for j in range(16):
                e = g * 16 + j
                v = v16[j]
                for k in range(_D // 16):
                    sl = pl.ds(k * 16, 16)
                    rows_v[e, sl] = rows_v[e, sl] * v
            return inner

        lax.fori_loop(0, _C // 16, _scale, 0)
        pltpu.sync_copy(rows_v, accum.at[row_v], add=True)
        return carry

    lax.fori_loop(0, _CPW, _chunk, 0)
    plsc.subcore_barrier()

    for j in range((nspan + _NS - 1) // _NS):
        idx = s + _NS * j

        @pl.when(idx < nspan)
        def _write_span():
            off = pl.multiple_of(idx * _C, 8)
            pltpu.sync_copy(accum.at[pl.ds(off, _C)],
                            out_hbm.at[c, pl.ds(off, _C)])

    @pl.when(s == _NS - 1)
    def _write_tail():
        pltpu.sync_copy(accum.at[pl.ds(nspan * _C, tail)],
                        out_hbm.at[c, pl.ds(nspan * _C, tail)])


def kernel(x, edge_index, edge_vals, W):
    feat = _feats(x, W).reshape(_HOPS * _N, _D)
    hop_off = (jnp.arange(_HOPS, dtype=jnp.int32) * _N)[:, None]
    col = (edge_index[:, 1, :] + hop_off).reshape(-1)
    row = edge_index[:, 0, :].reshape(-1)
    val = edge_vals.reshape(-1)
    pad = _EPAD - _TOTAL_E
    col = jnp.concatenate([col, jnp.zeros((pad,), jnp.int32)])
    row = jnp.concatenate([row, jnp.zeros((pad,), jnp.int32)])
    val = jnp.concatenate([val, jnp.zeros((pad,), jnp.float32)])
    partial = _propagate(feat,
                         col.reshape(_NW * _CPW, _C),
                         row.reshape(_NW * _CPW, _C),
                         val.reshape(_NW * _CPW, _C))
    return _combine(partial)


# speedup vs baseline: 9.5754x; 1.6513x over previous
import functools

import jax
import jax.numpy as jnp
from jax import lax
from jax.experimental import pallas as pl
from jax.experimental.pallas import tpu as pltpu
from jax.experimental.pallas import tpu_sc as plsc

_N = 10000
_E = 640000
_D = 128
_HOPS = 3
_NC = 2
_NS = 16
_NW = _NC * _NS
_C = 128
_TOTAL_E = _HOPS * _E
_CPW = 470
_NCHUNK = _NW * _CPW
_EPAD = _NCHUNK * _C
_MB = 2000


def _matmul_body(x_ref, w_ref, out_ref):
    out_ref[0] = jnp.dot(x_ref[...], w_ref[0],
                         preferred_element_type=jnp.float32)


def _feats(x, W):
    return pl.pallas_call(
        _matmul_body,
        grid=(_HOPS, _N // _MB),
        in_specs=[
            pl.BlockSpec((_MB, _D), lambda h, i: (i, 0)),
            pl.BlockSpec((1, _D, _D), lambda h, i: (h, 0, 0)),
        ],
        out_specs=pl.BlockSpec((1, _MB, _D), lambda h, i: (h, i, 0)),
        out_shape=jax.ShapeDtypeStruct((_HOPS, _N, _D), jnp.float32),
    )(x, W)


def _combine_body(p_ref, o_ref):
    o_ref[...] = jnp.maximum(p_ref[0] + p_ref[1], 0.0)


def _combine(p):
    return pl.pallas_call(
        _combine_body,
        grid=(_N // _MB,),
        in_specs=[pl.BlockSpec((_NC, _MB, _D), lambda i: (0, i, 0))],
        out_specs=pl.BlockSpec((_MB, _D), lambda i: (i, 0)),
        out_shape=jax.ShapeDtypeStruct((_N, _D), jnp.float32),
    )(p)


@functools.partial(
    pl.kernel,
    out_type=jax.ShapeDtypeStruct((_NC, _N, _D), jnp.float32),
    mesh=plsc.VectorSubcoreMesh(core_axis_name="c", subcore_axis_name="s"),
    scratch_types=[
        pltpu.VMEM((2, _C), jnp.int32),
        pltpu.VMEM((2, _C), jnp.int32),
        pltpu.VMEM((_C,), jnp.float32),
        pltpu.VMEM((_C,), jnp.float32),
        pltpu.VMEM((_C, _D), jnp.float32),
        pltpu.VMEM((_C, _D), jnp.float32),
        pltpu.VMEM_SHARED((_N, _D), jnp.float32),
        pltpu.SemaphoreType.DMA,
        pltpu.SemaphoreType.DMA,
        pltpu.SemaphoreType.DMA,
        pltpu.SemaphoreType.DMA,
    ],
)
def _propagate(feat_hbm, crv_hbm, vals_hbm, out_hbm,
               crv0, crv1, val0, val1, rows0, rows1, accum,
               semg0, semg1, sema0, sema1):
    c = lax.axis_index("c")
    s = lax.axis_index("s")
    w = s * _NC + c
    base = w * _CPW

    def _zrow(e, carry):
        for k in range(_D // 16):
            rows0[e, pl.ds(k * 16, 16)] = jnp.zeros((16,), jnp.float32)
        return carry

    lax.fori_loop(0, _C, _zrow, 0)

    nspan = _N // _C
    tail = _N - nspan * _C
    for j in range((nspan + _NS - 1) // _NS):
        idx = s + _NS * j

        @pl.when(idx < nspan)
        def _zero_span():
            off = pl.multiple_of(idx * _C, 8)
            pltpu.sync_copy(rows0, accum.at[pl.ds(off, _C)])

    @pl.when(s == _NS - 1)
    def _zero_tail():
        pltpu.sync_copy(rows0.at[pl.ds(0, tail)],
                        accum.at[pl.ds(nspan * _C, tail)])

    plsc.subcore_barrier()

    def _load_idx(g, crv_v, val_v):
        pltpu.sync_copy(crv_hbm.at[g], crv_v)
        pltpu.sync_copy(vals_hbm.at[g], val_v)

    def _start_gather(crv_v, rows_v, sem):
        pltpu.async_copy(feat_hbm.at[crv_v.at[0]], rows_v, sem)

    def _wait_gather(crv_v, rows_v, sem):
        pltpu.make_async_copy(feat_hbm.at[crv_v.at[0]], rows_v, sem).wait()

    def _start_scatter(crv_v, rows_v, sem):
        pltpu.async_copy(rows_v, accum.at[crv_v.at[1]], sem, add=True)

    def _wait_scatter(crv_v, rows_v, sem):
        pltpu.make_async_copy(rows_v, accum.at[crv_v.at[1]], sem).wait()

    def _scale(rows_v, val_v):
        def _grp(g, inner):
            v16 = val_v[pl.ds(g * 16, 16)]
            for j in range(16):
                e = g * 16 + j
                v = v16[j]
                for k in range(_D // 16):
                    sl = pl.ds(k * 16, 16)
                    rows_v[e, sl] = rows_v[e, sl] * v
            return inner

        lax.fori_loop(0, _C // 16, _grp, 0)

    _load_idx(base, crv0, val0)
    _start_gather(crv0, rows0, semg0)

    def _pair(t, carry):
        i1 = 2 * t + 1
        i2 = 2 * t + 2

        @pl.when(t > 0)
        def _():
            _wait_scatter(crv1, rows1, sema1)

        _load_idx(base + i1, crv1, val1)
        _start_gather(crv1, rows1, semg1)

        _wait_gather(crv0, rows0, semg0)
        _scale(rows0, val0)
        _start_scatter(crv0, rows0, sema0)

        @pl.when(i2 < _CPW)
        def _():
            _wait_scatter(crv0, rows0, sema0)
            _load_idx(base + i2, crv0, val0)
            _start_gather(crv0, rows0, semg0)

        _wait_gather(crv1, rows1, semg1)
        _scale(rows1, val1)
        _start_scatter(crv1, rows1, sema1)
        return carry

    lax.fori_loop(0, _CPW // 2, _pair, 0)
    _wait_scatter(crv0, rows0, sema0)
    _wait_scatter(crv1, rows1, sema1)

    plsc.subcore_barrier()

    for j in range((nspan + _NS - 1) // _NS):
        idx = s + _NS * j

        @pl.when(idx < nspan)
        def _write_span():
            off = pl.multiple_of(idx * _C, 8)
            pltpu.sync_copy(accum.at[pl.ds(off, _C)],
                            out_hbm.at[c, pl.ds(off, _C)])

    @pl.when(s == _NS - 1)
    def _write_tail():
        pltpu.sync_copy(accum.at[pl.ds(nspan * _C, tail)],
                        out_hbm.at[c, pl.ds(nspan * _C, tail)])


def kernel(x, edge_index, edge_vals, W):
    feat = _feats(x, W).reshape(_HOPS * _N, _D)
    hop_off = (jnp.arange(_HOPS, dtype=jnp.int32) * _N)[:, None]
    col = (edge_index[:, 1, :] + hop_off).reshape(-1)
    row = edge_index[:, 0, :].reshape(-1)
    val = edge_vals.reshape(-1)
    pad = _EPAD - _TOTAL_E
    pad_ar = jnp.arange(pad, dtype=jnp.int32)
    col = jnp.concatenate([col, pad_ar % (_HOPS * _N)])
    row = jnp.concatenate([row, pad_ar % _N])
    val = jnp.concatenate([val, jnp.zeros((pad,), jnp.float32)])
    crv = jnp.stack([col.reshape(_NCHUNK, _C),
                     row.reshape(_NCHUNK, _C)], axis=1)
    partial = _propagate(feat, crv, val.reshape(_NCHUNK, _C))
    return _combine(partial)
